# baseline (speedup 1.0000x reference)
import jax
import jax.numpy as jnp
from jax.experimental import pallas as pl
from jax.experimental.pallas import tpu as pltpu

_STRIDE = 8.0
_ANCHORS = ((10.0, 13.0), (16.0, 30.0), (33.0, 23.0))
_A = 3
_C = 85
_F = 76
_P = _F * _F
_T = 304
_KS = _P // _T
_N = 16
_R = _T * _N


def _body(x_ref, w_ref, b_ref, o_ref):
    w = w_ref[...]
    b = b_ref[...]

    xall = x_ref[...].reshape(_R, 128)
    acc = jax.lax.dot_general(
        xall, w, (((1,), (1,)), ((), ())),
        preferred_element_type=jnp.float32,
    ) + b

    base = pl.program_id(0) * _T
    rows = jax.lax.broadcasted_iota(jnp.int32, (_R, 128), 0)
    pix = rows // _N + base
    col = jax.lax.broadcasted_iota(jnp.int32, (_R, 128), 1)
    xs8 = (pix % _F).astype(jnp.float32) * _STRIDE
    ys8 = (pix // _F).astype(jnp.float32) * _STRIDE
    shift8 = jnp.where(col == 0, xs8, jnp.where(col == 1, ys8, 0.0))
    m_wh = (col == 2) | (col == 3)
    sign = jnp.where(m_wh, 1.0, -1.0)
    scale = jnp.where(col < 2, _STRIDE, 1.0)

    for a in range(_A):
        t = acc[:, a * 128:(a + 1) * 128]
        e = jnp.exp(t * sign)
        sig = 1.0 / (1.0 + e)
        aw, ah = _ANCHORS[a]
        anch = jnp.where(col == 2, aw, ah)
        val = jnp.where(m_wh, e * anch, sig * scale + shift8)
        o_ref[a, :, :, :] = val.reshape(_T, _N, 128)[:, :, :_C]


def kernel(xin, W, b):
    N = xin.shape[0]
    xt = jnp.transpose(xin, (2, 3, 0, 1)).reshape(_P, N, 128)
    w3 = W.reshape(_A, _C, 128)
    wp = jnp.pad(w3, ((0, 0), (0, 128 - _C), (0, 0))).reshape(_A * 128, 128)
    bp = jnp.pad(b.reshape(_A, _C), ((0, 0), (0, 128 - _C))).reshape(1, _A * 128)

    out = pl.pallas_call(
        _body,
        grid=(_KS,),
        in_specs=[
            pl.BlockSpec((_T, N, 128), lambda k: (k, 0, 0)),
            pl.BlockSpec((_A * 128, 128), lambda k: (0, 0)),
            pl.BlockSpec((1, _A * 128), lambda k: (0, 0)),
        ],
        out_specs=pl.BlockSpec((_A, _T, N, _C), lambda k: (0, k, 0, 0)),
        out_shape=jax.ShapeDtypeStruct((_A, _P, N, _C), jnp.float32),
        compiler_params=pltpu.CompilerParams(
            dimension_semantics=("parallel",),
        ),
    )(xt, wp, bp)
    return out.transpose(2, 0, 1, 3).reshape(N, _A * _P, _C)


# speedup vs baseline: 1.0315x; 1.0315x over previous
import jax
import jax.numpy as jnp
from jax.experimental import pallas as pl
from jax.experimental.pallas import tpu as pltpu

_STRIDE = 8.0
_ANCHORS = ((10.0, 13.0), (16.0, 30.0), (33.0, 23.0))
_A = 3
_C = 85
_F = 76
_P = _F * _F
_T = 304
_KS = _P // _T
_N = 16


def _body(x_ref, w_ref, bs_ref, o_ref):
    w = w_ref[...]
    bs = bs_ref[...]

    lane = jax.lax.broadcasted_iota(jnp.int32, (1, 128), 1)
    m_wh = (lane == 2) | (lane == 3)
    sign = jnp.where(m_wh, 1.0, -1.0)
    scale = jnp.where(lane < 2, _STRIDE, 1.0)
    anchs = [jnp.where(lane == 2, aw, ah) for aw, ah in _ANCHORS]

    base = pl.program_id(0) * _T
    rows = jax.lax.broadcasted_iota(jnp.int32, (_T, 128), 0) + base
    col = jax.lax.broadcasted_iota(jnp.int32, (_T, 128), 1)
    xs8 = (rows % _F).astype(jnp.float32) * _STRIDE
    ys8 = (rows // _F).astype(jnp.float32) * _STRIDE
    shift8 = jnp.where(col == 0, xs8, jnp.where(col == 1, ys8, 0.0))

    for n in range(_N):
        xb = x_ref[:, n * 128:(n + 1) * 128]
        acc = jax.lax.dot_general(
            xb, w, (((1,), (1,)), ((), ())),
            preferred_element_type=jnp.float32,
        )
        for a in range(_A):
            t = acc[:, a * 128:(a + 1) * 128]
            e = jnp.exp(t * sign + bs[:, a * 128:(a + 1) * 128])
            sig = 1.0 / (1.0 + e)
            val = jnp.where(m_wh, e * anchs[a], sig * scale + shift8)
            o_ref[n, a, :, :] = val[:, :_C]


def kernel(xin, W, b):
    N = xin.shape[0]
    xt = jnp.transpose(xin, (2, 3, 0, 1)).reshape(_P, N * 128)
    w3 = W.reshape(_A, _C, 128)
    wp = jnp.pad(w3, ((0, 0), (0, 128 - _C), (0, 0))).reshape(_A * 128, 128)
    bp = jnp.pad(b.reshape(_A, _C), ((0, 0), (0, 128 - _C))).reshape(1, _A * 128)
    sign = jnp.where((jnp.arange(128) == 2) | (jnp.arange(128) == 3), 1.0, -1.0)
    bsp = bp * jnp.tile(sign, _A)[None, :]

    out = pl.pallas_call(
        _body,
        grid=(_KS,),
        in_specs=[
            pl.BlockSpec((_T, N * 128), lambda k: (k, 0)),
            pl.BlockSpec((_A * 128, 128), lambda k: (0, 0)),
            pl.BlockSpec((1, _A * 128), lambda k: (0, 0)),
        ],
        out_specs=pl.BlockSpec((N, _A, _T, _C), lambda k: (0, 0, k, 0)),
        out_shape=jax.ShapeDtypeStruct((N, _A, _P, _C), jnp.float32),
        compiler_params=pltpu.CompilerParams(
            dimension_semantics=("parallel",),
        ),
    )(xt, wp, bsp)
    return out.reshape(N, _A * _P, _C)


# speedup vs baseline: 1.1650x; 1.1295x over previous
import jax
import jax.numpy as jnp
from jax.experimental import pallas as pl
from jax.experimental.pallas import tpu as pltpu

_STRIDE = 8.0
_ANCHORS = ((10.0, 13.0), (16.0, 30.0), (33.0, 23.0))
_A = 3
_C = 85
_F = 76
_P = _F * _F
_T = 304
_KS = _P // _T
_N = 16


def _body(x_ref, w_ref, bs_ref, o_ref):
    w = w_ref[...]
    bs = bs_ref[...]

    lane = jax.lax.broadcasted_iota(jnp.int32, (1, 128), 1)
    m_wh = (lane == 2) | (lane == 3)
    sign = jnp.where(m_wh, 1.0, -1.0)
    scale = jnp.where(lane < 2, _STRIDE, 1.0)
    anchs = [jnp.where(lane == 2, aw, ah) for aw, ah in _ANCHORS]

    base = pl.program_id(0) * _T
    rows = jax.lax.broadcasted_iota(jnp.int32, (_T, 128), 0) + base
    col = jax.lax.broadcasted_iota(jnp.int32, (_T, 128), 1)
    xs8 = (rows % _F).astype(jnp.float32) * _STRIDE
    ys8 = (rows // _F).astype(jnp.float32) * _STRIDE
    shift8 = jnp.where(col == 0, xs8, jnp.where(col == 1, ys8, 0.0))

    for n in range(_N):
        xb = x_ref[:, n, :]
        acc = jax.lax.dot_general(
            xb, w, (((1,), (1,)), ((), ())),
            preferred_element_type=jnp.float32,
        )
        for a in range(_A):
            t = acc[:, a * 128:(a + 1) * 128]
            e = jnp.exp(t * sign + bs[:, a * 128:(a + 1) * 128])
            sig = 1.0 / (1.0 + e)
            val = jnp.where(m_wh, e * anchs[a], sig * scale + shift8)
            o_ref[n, a, :, :] = val[:, :_C]


def kernel(xin, W, b):
    N = xin.shape[0]
    xt = jnp.transpose(xin, (2, 3, 0, 1)).reshape(_P, N, 128)
    w3 = W.reshape(_A, _C, 128)
    wp = jnp.pad(w3, ((0, 0), (0, 128 - _C), (0, 0))).reshape(_A * 128, 128)
    bp = jnp.pad(b.reshape(_A, _C), ((0, 0), (0, 128 - _C))).reshape(1, _A * 128)
    sign = jnp.where((jnp.arange(128) == 2) | (jnp.arange(128) == 3), 1.0, -1.0)
    bsp = bp * jnp.tile(sign, _A)[None, :]

    out = pl.pallas_call(
        _body,
        grid=(_KS,),
        in_specs=[
            pl.BlockSpec((_T, N, 128), lambda k: (k, 0, 0)),
            pl.BlockSpec((_A * 128, 128), lambda k: (0, 0)),
            pl.BlockSpec((1, _A * 128), lambda k: (0, 0)),
        ],
        out_specs=pl.BlockSpec((N, _A, _T, _C), lambda k: (0, 0, k, 0)),
        out_shape=jax.ShapeDtypeStruct((N, _A, _P, _C), jnp.float32),
        compiler_params=pltpu.CompilerParams(
            dimension_semantics=("parallel",),
        ),
    )(xt, wp, bsp)
    return out.reshape(N, _A * _P, _C)
